# baseline (speedup 1.0000x reference)
import functools

import jax
import jax.numpy as jnp
from jax import lax
from jax.experimental import pallas as pl
from jax.experimental.pallas import tpu as pltpu
from jax.experimental.pallas import tpu_sc as plsc

NUM_USERS = 1000000
EMBED_DIM = 64
BATCH = 16384

NC = 2
NS = 16
L = 16
NW = NC * NS
B_PER_W = BATCH // NW
CW = 256
N_FULL = NUM_USERS // CW
TAIL_LO = N_FULL * CW
TAIL_W = NUM_USERS - TAIL_LO
CHUNKS_PER_W = N_FULL // NW
EXTRA = N_FULL % NW
NVEC = BATCH // L
HALF = B_PER_W // 2


def _extract_body(ids_hbm, tabT_hbm, tail_hbm, scr_hbm,
                  ids_v, mid_v, mpos_v, chunk_v, stage_v, posl_v, sem, sem2):
    wid = lax.axis_index("s") * NC + lax.axis_index("c")
    c_start = CHUNKS_PER_W * wid + jnp.minimum(wid, EXTRA)
    n_chunks = CHUNKS_PER_W + jnp.where(wid < EXTRA, 1, 0)

    pltpu.sync_copy(ids_hbm, ids_v)

    is_last = wid == NW - 1
    tl_lo = CW * c_start
    tl_hi = jnp.where(is_last, NUM_USERS, CW * (c_start + n_chunks))

    def binify(j, off):
        v = ids_v[pl.ds(j * L, L)]
        pos16 = j * L + lax.iota(jnp.int32, L)
        m = (v >= tl_lo) & (v < tl_hi)
        plsc.store_compressed(mid_v.at[pl.ds(off, L)], v, mask=m)
        plsc.store_compressed(mpos_v.at[pl.ds(off, L)], pos16, mask=m)
        return off + plsc.all_reduce_population_count(m)[0]

    off = lax.fori_loop(0, NVEC, binify, jnp.int32(0))
    mid_v[pl.ds(off, L)] = jnp.full((L,), -1, jnp.int32)
    mpos_v[pl.ds(off, L)] = jnp.full((L,), -1, jnp.int32)
    nmv = (off + L - 1) // L

    def scan_and_extract(lo, hi):
        def scan(j, _):
            v = mid_v[pl.ds(j * L, L)]
            p = mpos_v[pl.ds(j * L, L)]
            m = (v >= lo) & (v < hi)

            @pl.when(jnp.any(m))
            def _extract16():
                cols = jnp.where(m, v - lo, 0)
                posx = jnp.where(m, p, -1)
                lanes = lax.iota(jnp.int32, L)
                for c in range(EMBED_DIM):
                    cc = jnp.full((L,), c, jnp.int32)
                    vals = plsc.load_gather(chunk_v, [cc, cols])
                    plsc.store_scatter(stage_v, [lanes, cc], vals)
                posl_v[pl.ds(0, L)] = posx
                pltpu.async_copy(
                    stage_v,
                    scr_hbm.at[plsc.Indices(posl_v.at[pl.ds(0, L)],
                                            ignored_value=-1)],
                    sem2).wait()

            return _

        lax.fori_loop(0, nmv, scan, None)

    def do_chunk(ci_local, _):
        lo = pl.multiple_of(CW * (c_start + ci_local), 128)
        pltpu.async_copy(tabT_hbm.at[:, pl.ds(lo, CW)], chunk_v, sem).wait()
        scan_and_extract(lo, lo + CW)
        return _

    lax.fori_loop(0, n_chunks, do_chunk, None)

    @pl.when(is_last)
    def _tail():
        pltpu.async_copy(tail_hbm, chunk_v.at[:, pl.ds(0, 128)], sem).wait()
        scan_and_extract(TAIL_LO, NUM_USERS)


def _dot_body(scru_hbm, scrp_hbm, out_hbm, urows_v, prows_v, out_v, sem):
    wid = lax.axis_index("s") * NC + lax.axis_index("c")
    base = wid * B_PER_W

    for h in range(2):
        cpu = pltpu.async_copy(
            scru_hbm.at[pl.ds(base + h * HALF, HALF), :], urows_v, sem)
        cpp = pltpu.async_copy(
            scrp_hbm.at[pl.ds(base + h * HALF, HALF), :], prows_v, sem)
        cpu.wait()
        cpp.wait()

        def group(g, _):
            row = g * L + lax.iota(jnp.int32, L)
            acc = jnp.zeros((L,), jnp.float32)
            for d in range(EMBED_DIM):
                cc = jnp.full((L,), d, jnp.int32)
                u = plsc.load_gather(urows_v, [row, cc])
                p = plsc.load_gather(prows_v, [row, cc])
                acc = acc + u * p
            out_v[pl.ds(h * HALF + g * L, L)] = 1.0 / (1.0 + jnp.exp(-acc))
            return _

        lax.fori_loop(0, HALF // L, group, None)

    pltpu.sync_copy(out_v, out_hbm.at[pl.ds(base, B_PER_W)])


def _make_extract():
    mesh = plsc.VectorSubcoreMesh(core_axis_name="c", subcore_axis_name="s")
    return pl.kernel(
        _extract_body,
        out_type=jax.ShapeDtypeStruct((BATCH, 128), jnp.float32),
        mesh=mesh,
        scratch_types=[
            pltpu.VMEM((BATCH,), jnp.int32),
            pltpu.VMEM((BATCH + L,), jnp.int32),
            pltpu.VMEM((BATCH + L,), jnp.int32),
            pltpu.VMEM((EMBED_DIM, CW), jnp.float32),
            pltpu.VMEM((L, 128), jnp.float32),
            pltpu.VMEM((L,), jnp.int32),
            pltpu.SemaphoreType.DMA,
            pltpu.SemaphoreType.DMA,
        ],
        compiler_params=pltpu.CompilerParams(needs_layout_passes=False),
    )


def _make_dot():
    mesh = plsc.VectorSubcoreMesh(core_axis_name="c", subcore_axis_name="s")
    return pl.kernel(
        _dot_body,
        out_type=jax.ShapeDtypeStruct((BATCH,), jnp.float32),
        mesh=mesh,
        scratch_types=[
            pltpu.VMEM((HALF, 128), jnp.float32),
            pltpu.VMEM((HALF, 128), jnp.float32),
            pltpu.VMEM((B_PER_W,), jnp.float32),
            pltpu.SemaphoreType.DMA,
        ],
        compiler_params=pltpu.CompilerParams(needs_layout_passes=False),
    )


@jax.jit
def _run(user_ids, partner_ids, user_embed, partner_embed):
    extract = _make_extract()
    utail = jnp.pad(user_embed.T[:, TAIL_LO:], ((0, 0), (0, 128 - TAIL_W)))
    ptail = jnp.pad(partner_embed.T[:, TAIL_LO:], ((0, 0), (0, 128 - TAIL_W)))
    scru = extract(user_ids, user_embed.T, utail)
    scrp = extract(partner_ids, partner_embed.T, ptail)
    return _make_dot()(scru, scrp)


def kernel(user_ids, partner_ids, user_embed, partner_embed):
    return _run(user_ids.astype(jnp.int32), partner_ids.astype(jnp.int32),
                user_embed, partner_embed)


# speedup vs baseline: 4.1418x; 4.1418x over previous
import functools

import jax
import jax.numpy as jnp
from jax import lax
from jax.experimental import pallas as pl
from jax.experimental.pallas import tpu as pltpu
from jax.experimental.pallas import tpu_sc as plsc

NUM_USERS = 1000000
EMBED_DIM = 64
BATCH = 16384

NC = 2
NS = 16
L = 16
NW = NC * NS
B_PER_W = BATCH // NW
CW = 512
N_FULL = NUM_USERS // CW
TAIL_LO = N_FULL * CW
TAIL_W = NUM_USERS - TAIL_LO
CHUNKS_PER_W = N_FULL // NW
EXTRA = N_FULL % NW
NVEC = BATCH // L
HALF = B_PER_W // 2
PK = 16384
STAGE = 112
FLUSH = STAGE - L


def _extract_body(ids_hbm, tabT_hbm, tail_hbm, scr_hbm,
                  ids_v, mine_v, mm_v, chunk0_v, chunk1_v, stage_v, posl_v,
                  sem0, sem1, sem2):
    wid = lax.axis_index("s") * NC + lax.axis_index("c")
    c_start = CHUNKS_PER_W * wid + jnp.minimum(wid, EXTRA)
    n_chunks = CHUNKS_PER_W + jnp.where(wid < EXTRA, 1, 0)

    pltpu.sync_copy(ids_hbm, ids_v)
    for q in range(STAGE // L):
        posl_v[pl.ds(q * L, L)] = jnp.full((L,), -1, jnp.int32)

    is_last = wid == NW - 1
    tl_lo = CW * c_start
    tl_hi = jnp.where(is_last, NUM_USERS, CW * (c_start + n_chunks))

    def binify(j, off):
        v = ids_v[pl.ds(j * L, L)]
        rel = v - tl_lo
        pos16 = j * L + lax.iota(jnp.int32, L)
        m = (rel >= 0) & (v < tl_hi)
        plsc.store_compressed(mine_v.at[pl.ds(off, L)], rel * PK + pos16,
                              mask=m)
        return off + plsc.all_reduce_population_count(m)[0]

    noff = lax.fori_loop(0, NVEC, binify, jnp.int32(0))
    mine_v[pl.ds(noff, L)] = jnp.full((L,), -1, jnp.int32)
    nmv = (noff + L - 1) // L

    def flush(soff):
        @pl.when(soff >= FLUSH)
        def _do():
            pltpu.async_copy(
                stage_v,
                scr_hbm.at[plsc.Indices(posl_v.at[pl.ds(0, STAGE)],
                                        ignored_value=-1)],
                sem2).wait()
            for q in range(STAGE // L):
                posl_v[pl.ds(q * L, L)] = jnp.full((L,), -1, jnp.int32)

        return jnp.where(soff >= FLUSH, 0, soff)

    def process(chunk_v, lo, soff):
        lo_rel = lo - tl_lo

        def scan(j, cnt):
            pck = mine_v[pl.ds(j * L, L)]
            rel = lax.shift_right_logical(pck, 14)
            m = (pck >= 0) & (rel >= lo_rel) & (rel < lo_rel + CW)
            plsc.store_compressed(mm_v.at[pl.ds(cnt, L)], pck, mask=m)
            return cnt + plsc.all_reduce_population_count(m)[0]

        cnt = lax.fori_loop(0, nmv, scan, jnp.int32(0))
        mm_v[pl.ds(cnt, L)] = jnp.full((L,), -1, jnp.int32)

        def group(g, soff):
            soff = flush(soff)
            pck = mm_v[pl.ds(g * L, L)]
            valid = pck >= 0
            cols = jnp.where(valid, lax.shift_right_logical(pck, 14) - lo_rel,
                             0)
            posx = jnp.where(valid, pck & (PK - 1), -1)
            lanes = soff + lax.iota(jnp.int32, L)
            for c in range(EMBED_DIM):
                cc = jnp.full((L,), c, jnp.int32)
                vals = plsc.load_gather(chunk_v, [cc, cols])
                plsc.store_scatter(stage_v, [lanes, cc], vals)
            posl_v[pl.ds(soff, L)] = posx
            return soff + L

        return lax.fori_loop(0, (cnt + L - 1) // L, group, soff)

    def lo_of(i):
        return pl.multiple_of(
            CW * (c_start + jnp.minimum(i, n_chunks - 1)), 128)

    pltpu.async_copy(tabT_hbm.at[:, pl.ds(lo_of(0), CW)], chunk0_v, sem0)

    def pair(k, soff):
        i0 = 2 * k
        pltpu.async_copy(tabT_hbm.at[:, pl.ds(lo_of(i0 + 1), CW)],
                         chunk1_v, sem1)
        pltpu.make_async_copy(tabT_hbm.at[:, pl.ds(0, CW)], chunk0_v,
                              sem0).wait()
        soff = process(chunk0_v, lo_of(i0), soff)
        pltpu.async_copy(tabT_hbm.at[:, pl.ds(lo_of(i0 + 2), CW)],
                         chunk0_v, sem0)
        pltpu.make_async_copy(tabT_hbm.at[:, pl.ds(0, CW)], chunk1_v,
                              sem1).wait()
        soff = process(chunk1_v, lo_of(i0 + 1), soff)
        return soff

    soff = lax.fori_loop(0, (n_chunks + 1) // 2, pair, jnp.int32(0))
    pltpu.make_async_copy(tabT_hbm.at[:, pl.ds(0, CW)], chunk0_v, sem0).wait()

    def final(soff):
        @pl.when(soff > 0)
        def _do():
            pltpu.async_copy(
                stage_v,
                scr_hbm.at[plsc.Indices(posl_v.at[pl.ds(0, STAGE)],
                                        ignored_value=-1)],
                sem2).wait()

    @pl.when(is_last)
    def _tail():
        pltpu.sync_copy(tail_hbm, chunk0_v.at[:, pl.ds(0, 128)])
        final(process(chunk0_v, TAIL_LO, soff))

    @pl.when(jnp.logical_not(is_last))
    def _no_tail():
        final(soff)


def _dot_body(scru_hbm, scrp_hbm, out_hbm, urows_v, prows_v, out_v, sem):
    wid = lax.axis_index("s") * NC + lax.axis_index("c")
    base = wid * B_PER_W

    for h in range(2):
        cpu = pltpu.async_copy(
            scru_hbm.at[pl.ds(base + h * HALF, HALF), :], urows_v, sem)
        cpp = pltpu.async_copy(
            scrp_hbm.at[pl.ds(base + h * HALF, HALF), :], prows_v, sem)
        cpu.wait()
        cpp.wait()

        def group(g, _):
            row = g * L + lax.iota(jnp.int32, L)
            acc = jnp.zeros((L,), jnp.float32)
            for d in range(EMBED_DIM):
                cc = jnp.full((L,), d, jnp.int32)
                u = plsc.load_gather(urows_v, [row, cc])
                p = plsc.load_gather(prows_v, [row, cc])
                acc = acc + u * p
            out_v[pl.ds(h * HALF + g * L, L)] = 1.0 / (1.0 + jnp.exp(-acc))
            return _

        lax.fori_loop(0, HALF // L, group, None)

    pltpu.sync_copy(out_v, out_hbm.at[pl.ds(base, B_PER_W)])


def _make_extract():
    mesh = plsc.VectorSubcoreMesh(core_axis_name="c", subcore_axis_name="s")
    return pl.kernel(
        _extract_body,
        out_type=jax.ShapeDtypeStruct((BATCH, 128), jnp.float32),
        mesh=mesh,
        scratch_types=[
            pltpu.VMEM((BATCH,), jnp.int32),
            pltpu.VMEM((BATCH + L,), jnp.int32),
            pltpu.VMEM((BATCH + L,), jnp.int32),
            pltpu.VMEM((EMBED_DIM, CW), jnp.float32),
            pltpu.VMEM((EMBED_DIM, CW), jnp.float32),
            pltpu.VMEM((STAGE, 128), jnp.float32),
            pltpu.VMEM((STAGE,), jnp.int32),
            pltpu.SemaphoreType.DMA,
            pltpu.SemaphoreType.DMA,
            pltpu.SemaphoreType.DMA,
        ],
        compiler_params=pltpu.CompilerParams(needs_layout_passes=False),
    )


def _make_dot():
    mesh = plsc.VectorSubcoreMesh(core_axis_name="c", subcore_axis_name="s")
    return pl.kernel(
        _dot_body,
        out_type=jax.ShapeDtypeStruct((BATCH,), jnp.float32),
        mesh=mesh,
        scratch_types=[
            pltpu.VMEM((HALF, 128), jnp.float32),
            pltpu.VMEM((HALF, 128), jnp.float32),
            pltpu.VMEM((B_PER_W,), jnp.float32),
            pltpu.SemaphoreType.DMA,
        ],
        compiler_params=pltpu.CompilerParams(needs_layout_passes=False),
    )


@jax.jit
def _run(user_ids, partner_ids, user_embed, partner_embed):
    extract = _make_extract()
    utail = jnp.pad(user_embed.T[:, TAIL_LO:], ((0, 0), (0, 128 - TAIL_W)))
    ptail = jnp.pad(partner_embed.T[:, TAIL_LO:], ((0, 0), (0, 128 - TAIL_W)))
    scru = extract(user_ids, user_embed.T, utail)
    scrp = extract(partner_ids, partner_embed.T, ptail)
    return _make_dot()(scru, scrp)


def kernel(user_ids, partner_ids, user_embed, partner_embed):
    return _run(user_ids.astype(jnp.int32), partner_ids.astype(jnp.int32),
                user_embed, partner_embed)


# speedup vs baseline: 4.1574x; 1.0038x over previous
import functools

import jax
import jax.numpy as jnp
from jax import lax
from jax.experimental import pallas as pl
from jax.experimental.pallas import tpu as pltpu
from jax.experimental.pallas import tpu_sc as plsc

NUM_USERS = 1000000
EMBED_DIM = 64
BATCH = 16384

NC = 2
NS = 16
L = 16
NW = NC * NS
B_PER_W = BATCH // NW
CW = 512
N_FULL = NUM_USERS // CW
TAIL_LO = N_FULL * CW
TAIL_W = NUM_USERS - TAIL_LO
CHUNKS_PER_W = N_FULL // NW
EXTRA = N_FULL % NW
NVEC = BATCH // L
HALF = B_PER_W // 2
QUART = B_PER_W // 4
PK = 16384
STAGE = 112
FLUSH = STAGE - L


def _extract_body(ids_hbm, tabT_hbm, tail_hbm, scr_hbm,
                  ids_v, mine_v, mm_v, chunk0_v, chunk1_v, stage_v, posl_v,
                  sem0, sem1, sem2):
    wid = lax.axis_index("s") * NC + lax.axis_index("c")
    c_start = CHUNKS_PER_W * wid + jnp.minimum(wid, EXTRA)
    n_chunks = CHUNKS_PER_W + jnp.where(wid < EXTRA, 1, 0)

    pltpu.sync_copy(ids_hbm, ids_v)
    for q in range(STAGE // L):
        posl_v[pl.ds(q * L, L)] = jnp.full((L,), -1, jnp.int32)

    is_last = wid == NW - 1
    tl_lo = CW * c_start
    tl_hi = jnp.where(is_last, NUM_USERS, CW * (c_start + n_chunks))

    def binify(j, off):
        v = ids_v[pl.ds(j * L, L)]
        rel = v - tl_lo
        pos16 = j * L + lax.iota(jnp.int32, L)
        m = (rel >= 0) & (v < tl_hi)
        plsc.store_compressed(mine_v.at[pl.ds(off, L)], rel * PK + pos16,
                              mask=m)
        return off + plsc.all_reduce_population_count(m)[0]

    noff = lax.fori_loop(0, NVEC, binify, jnp.int32(0))
    mine_v[pl.ds(noff, L)] = jnp.full((L,), -1, jnp.int32)
    nmv = (noff + L - 1) // L

    def flush(soff):
        @pl.when(soff >= FLUSH)
        def _do():
            pltpu.async_copy(
                stage_v,
                scr_hbm.at[plsc.Indices(posl_v.at[pl.ds(0, STAGE)],
                                        ignored_value=-1)],
                sem2).wait()
            for q in range(STAGE // L):
                posl_v[pl.ds(q * L, L)] = jnp.full((L,), -1, jnp.int32)

        return jnp.where(soff >= FLUSH, 0, soff)

    def process(chunk_v, lo, soff):
        lo_rel = lo - tl_lo

        def scan(j, cnt):
            pck = mine_v[pl.ds(j * L, L)]
            rel = lax.shift_right_logical(pck, 14)
            m = (pck >= 0) & (rel >= lo_rel) & (rel < lo_rel + CW)
            plsc.store_compressed(mm_v.at[pl.ds(cnt, L)], pck, mask=m)
            return cnt + plsc.all_reduce_population_count(m)[0]

        cnt = lax.fori_loop(0, nmv, scan, jnp.int32(0))
        mm_v[pl.ds(cnt, L)] = jnp.full((L,), -1, jnp.int32)

        def group(g, soff):
            soff = flush(soff)
            pck = mm_v[pl.ds(g * L, L)]
            valid = pck >= 0
            cols = jnp.where(valid, lax.shift_right_logical(pck, 14) - lo_rel,
                             0)
            posx = jnp.where(valid, pck & (PK - 1), -1)
            lanes = soff + lax.iota(jnp.int32, L)
            for c in range(EMBED_DIM):
                cc = jnp.full((L,), c, jnp.int32)
                vals = plsc.load_gather(chunk_v, [cc, cols])
                plsc.store_scatter(stage_v, [lanes, cc], vals)
            posl_v[pl.ds(soff, L)] = posx
            return soff + L

        return lax.fori_loop(0, (cnt + L - 1) // L, group, soff)

    def lo_of(i):
        return pl.multiple_of(
            CW * (c_start + jnp.minimum(i, n_chunks - 1)), 128)

    pltpu.async_copy(tabT_hbm.at[:, pl.ds(lo_of(0), CW)], chunk0_v, sem0)

    def pair(k, soff):
        i0 = 2 * k
        pltpu.async_copy(tabT_hbm.at[:, pl.ds(lo_of(i0 + 1), CW)],
                         chunk1_v, sem1)
        pltpu.make_async_copy(tabT_hbm.at[:, pl.ds(0, CW)], chunk0_v,
                              sem0).wait()
        soff = process(chunk0_v, lo_of(i0), soff)
        pltpu.async_copy(tabT_hbm.at[:, pl.ds(lo_of(i0 + 2), CW)],
                         chunk0_v, sem0)
        pltpu.make_async_copy(tabT_hbm.at[:, pl.ds(0, CW)], chunk1_v,
                              sem1).wait()
        soff = process(chunk1_v, lo_of(i0 + 1), soff)
        return soff

    soff = lax.fori_loop(0, (n_chunks + 1) // 2, pair, jnp.int32(0))
    pltpu.make_async_copy(tabT_hbm.at[:, pl.ds(0, CW)], chunk0_v, sem0).wait()

    def final(soff):
        @pl.when(soff > 0)
        def _do():
            pltpu.async_copy(
                stage_v,
                scr_hbm.at[plsc.Indices(posl_v.at[pl.ds(0, STAGE)],
                                        ignored_value=-1)],
                sem2).wait()

    @pl.when(is_last)
    def _tail():
        pltpu.sync_copy(tail_hbm, chunk0_v.at[:, pl.ds(0, 128)])
        final(process(chunk0_v, TAIL_LO, soff))

    @pl.when(jnp.logical_not(is_last))
    def _no_tail():
        final(soff)


def _dot_body(scru_hbm, scrp_hbm, out_hbm,
              urows0_v, prows0_v, urows1_v, prows1_v, out_v, sem):
    wid = lax.axis_index("s") * NC + lax.axis_index("c")
    base = wid * B_PER_W

    bufs = ((urows0_v, prows0_v), (urows1_v, prows1_v))
    NQ = B_PER_W // QUART

    def issue(q):
        b = bufs[q % 2]
        return (pltpu.async_copy(
                    scru_hbm.at[pl.ds(base + q * QUART, QUART), :], b[0], sem),
                pltpu.async_copy(
                    scrp_hbm.at[pl.ds(base + q * QUART, QUART), :], b[1], sem))

    pend = [issue(0), issue(1)]
    for q in range(NQ):
        pend[q % 2][0].wait()
        pend[q % 2][1].wait()
        urows_v, prows_v = bufs[q % 2]

        def group(g, _, urows_v=urows_v, prows_v=prows_v, q=q):
            row = g * L + lax.iota(jnp.int32, L)
            acc = jnp.zeros((L,), jnp.float32)
            for d in range(EMBED_DIM):
                cc = jnp.full((L,), d, jnp.int32)
                u = plsc.load_gather(urows_v, [row, cc])
                p = plsc.load_gather(prows_v, [row, cc])
                acc = acc + u * p
            out_v[pl.ds(q * QUART + g * L, L)] = 1.0 / (1.0 + jnp.exp(-acc))
            return _

        lax.fori_loop(0, QUART // L, group, None)
        if q + 2 < NQ:
            pend[q % 2] = issue(q + 2)

    pltpu.sync_copy(out_v, out_hbm.at[pl.ds(base, B_PER_W)])


def _make_extract():
    mesh = plsc.VectorSubcoreMesh(core_axis_name="c", subcore_axis_name="s")
    return pl.kernel(
        _extract_body,
        out_type=jax.ShapeDtypeStruct((BATCH, 128), jnp.float32),
        mesh=mesh,
        scratch_types=[
            pltpu.VMEM((BATCH,), jnp.int32),
            pltpu.VMEM((BATCH + L,), jnp.int32),
            pltpu.VMEM((BATCH + L,), jnp.int32),
            pltpu.VMEM((EMBED_DIM, CW), jnp.float32),
            pltpu.VMEM((EMBED_DIM, CW), jnp.float32),
            pltpu.VMEM((STAGE, 128), jnp.float32),
            pltpu.VMEM((STAGE,), jnp.int32),
            pltpu.SemaphoreType.DMA,
            pltpu.SemaphoreType.DMA,
            pltpu.SemaphoreType.DMA,
        ],
        compiler_params=pltpu.CompilerParams(needs_layout_passes=False),
    )


def _make_dot():
    mesh = plsc.VectorSubcoreMesh(core_axis_name="c", subcore_axis_name="s")
    return pl.kernel(
        _dot_body,
        out_type=jax.ShapeDtypeStruct((BATCH,), jnp.float32),
        mesh=mesh,
        scratch_types=[
            pltpu.VMEM((QUART, 128), jnp.float32),
            pltpu.VMEM((QUART, 128), jnp.float32),
            pltpu.VMEM((QUART, 128), jnp.float32),
            pltpu.VMEM((QUART, 128), jnp.float32),
            pltpu.VMEM((B_PER_W,), jnp.float32),
            pltpu.SemaphoreType.DMA,
        ],
        compiler_params=pltpu.CompilerParams(needs_layout_passes=False),
    )


@jax.jit
def _run(user_ids, partner_ids, user_embed, partner_embed):
    extract = _make_extract()
    utail = jnp.pad(user_embed.T[:, TAIL_LO:], ((0, 0), (0, 128 - TAIL_W)))
    ptail = jnp.pad(partner_embed.T[:, TAIL_LO:], ((0, 0), (0, 128 - TAIL_W)))
    scru = extract(user_ids, user_embed.T, utail)
    scrp = extract(partner_ids, partner_embed.T, ptail)
    return _make_dot()(scru, scrp)


def kernel(user_ids, partner_ids, user_embed, partner_embed):
    return _run(user_ids.astype(jnp.int32), partner_ids.astype(jnp.int32),
                user_embed, partner_embed)
